# Initial kernel scaffold; baseline (speedup 1.0000x reference)
import jax
import jax.numpy as jnp
from jax.experimental import pallas as pl


def kernel(x, table):
    raise NotImplementedError("write your pallas kernel here")


# speedup vs baseline: 2.6132x; 2.6132x over previous
import functools
import math

import jax
import jax.numpy as jnp
from jax import lax
from jax.experimental import pallas as pl
from jax.experimental.pallas import tpu as pltpu
from jax.experimental.pallas import tpu_sc as plsc

_VOCAB = 100000
_D = 128
_SCALE = math.sqrt(float(_D))

_NC = 2
_NS = 16
_NW = _NC * _NS

_B = 4096 * 50
_ROWS_PER_W = _B // _NW
_C = 400
_NCHUNK = _ROWS_PER_W // _C


def _body(x_hbm, table_hbm, out_hbm, idx_v, rows_v, sem_g):
    c = lax.axis_index("c")
    s = lax.axis_index("s")
    wid = s * _NC + c
    base = wid * _ROWS_PER_W

    pltpu.sync_copy(x_hbm.at[pl.ds(base, _ROWS_PER_W)], idx_v)

    for ch in range(_NCHUNK):
        pltpu.async_copy(
            table_hbm.at[idx_v.at[pl.ds(ch * _C, _C)]], rows_v, sem_g
        ).wait()

        def row_body(r, _):
            for k in range(_D // 16):
                sl = rows_v[r, pl.ds(k * 16, 16)]
                rows_v[r, pl.ds(k * 16, 16)] = sl * _SCALE
            return _

        lax.fori_loop(0, _C, row_body, 0)

        pltpu.sync_copy(rows_v, out_hbm.at[pl.ds(base + ch * _C, _C)])


@functools.partial(jax.jit, static_argnames=())
def _sc_embed(x_flat, table):
    mesh = plsc.VectorSubcoreMesh(core_axis_name="c", subcore_axis_name="s")
    f = pl.kernel(
        _body,
        out_type=jax.ShapeDtypeStruct((_B, _D), jnp.float32),
        mesh=mesh,
        scratch_types=[
            pltpu.VMEM((_ROWS_PER_W,), jnp.int32),
            pltpu.VMEM((_C, _D), jnp.float32),
            pltpu.SemaphoreType.DMA,
        ],
    )
    return f(x_flat, table)


def kernel(x, table):
    x_flat = x.reshape(-1)
    out = _sc_embed(x_flat, table)
    return out.reshape(x.shape + (table.shape[1],))


# speedup vs baseline: 2.9215x; 1.1180x over previous
import functools
import math

import jax
import jax.numpy as jnp
from jax import lax
from jax.experimental import pallas as pl
from jax.experimental.pallas import tpu as pltpu
from jax.experimental.pallas import tpu_sc as plsc

_VOCAB = 100000
_D = 128
_SCALE = math.sqrt(float(_D))

_NC = 2
_NS = 16
_NW = _NC * _NS

_B = 4096 * 50
_ROWS_PER_W = _B // _NW
_C = 400
_NCHUNK = _ROWS_PER_W // _C


def _scale_chunk(rows_v):
    @plsc.parallel_loop(0, _C, step=1, unroll=4)
    def _(r):
        for k in range(_D // 16):
            sl = rows_v[r, pl.ds(k * 16, 16)]
            rows_v[r, pl.ds(k * 16, 16)] = sl * _SCALE


def _body(x_hbm, table_hbm, out_hbm, idx_v, rows0, rows1, sg0, sg1, so0, so1):
    c = lax.axis_index("c")
    s = lax.axis_index("s")
    wid = s * _NC + c
    base = wid * _ROWS_PER_W

    pltpu.sync_copy(x_hbm.at[pl.ds(base, _ROWS_PER_W)], idx_v)

    bufs = (rows0, rows1)
    gsems = (sg0, sg1)
    osems = (so0, so1)

    def start_gather(ch):
        b = ch % 2
        return pltpu.async_copy(
            table_hbm.at[idx_v.at[pl.ds(ch * _C, _C)]], bufs[b], gsems[b]
        )

    gathers = [None] * _NCHUNK
    outs = [None] * _NCHUNK
    gathers[0] = start_gather(0)

    for ch in range(_NCHUNK):
        b = ch % 2
        gathers[ch].wait()
        if ch + 1 < _NCHUNK:
            if ch >= 1:
                outs[ch - 1].wait()
            gathers[ch + 1] = start_gather(ch + 1)
        _scale_chunk(bufs[b])
        outs[ch] = pltpu.async_copy(
            bufs[b], out_hbm.at[pl.ds(base + ch * _C, _C)], osems[b]
        )

    outs[_NCHUNK - 2].wait()
    outs[_NCHUNK - 1].wait()


@jax.jit
def _sc_embed(x_flat, table):
    mesh = plsc.VectorSubcoreMesh(core_axis_name="c", subcore_axis_name="s")
    f = pl.kernel(
        _body,
        out_type=jax.ShapeDtypeStruct((_B, _D), jnp.float32),
        mesh=mesh,
        scratch_types=[
            pltpu.VMEM((_ROWS_PER_W,), jnp.int32),
            pltpu.VMEM((_C, _D), jnp.float32),
            pltpu.VMEM((_C, _D), jnp.float32),
            pltpu.SemaphoreType.DMA,
            pltpu.SemaphoreType.DMA,
            pltpu.SemaphoreType.DMA,
            pltpu.SemaphoreType.DMA,
        ],
    )
    return f(x_flat, table)


def kernel(x, table):
    x_flat = x.reshape(-1)
    out = _sc_embed(x_flat, table)
    return out.reshape(x.shape + (table.shape[1],))


# speedup vs baseline: 5.1125x; 1.7499x over previous
import functools
import math

import jax
import jax.numpy as jnp
from jax import lax
from jax.experimental import pallas as pl
from jax.experimental.pallas import tpu as pltpu
from jax.experimental.pallas import tpu_sc as plsc

_VOCAB = 100000
_D = 128
_SCALE = math.sqrt(float(_D))

_NC = 2
_NS = 16
_NW = _NC * _NS

_SENT = 4096
_L = 50
_B = _SENT * _L
_ROWS_PER_W = _B // _NW
_SENT_PER_W = _SENT // _NW
_CS = 8
_C = _CS * _L
_NCHUNK = _SENT_PER_W // _CS


def _scale_chunk(rows_v):
    @plsc.parallel_loop(0, _C, step=1, unroll=4)
    def _(r):
        for k in range(_D // 16):
            sl = rows_v[r, pl.ds(k * 16, 16)]
            rows_v[r, pl.ds(k * 16, 16)] = sl * _SCALE


def _body(x_hbm, table_hbm, out_hbm, idx_v, rows0, rows1, sg0, sg1, so0, so1):
    c = lax.axis_index("c")
    s = lax.axis_index("s")
    wid = s * _NC + c
    base = wid * _ROWS_PER_W
    sent0 = wid * _SENT_PER_W

    pltpu.sync_copy(x_hbm.at[pl.ds(base, _ROWS_PER_W)], idx_v)

    bufs = (rows0, rows1)
    gsems = (sg0, sg1)
    osems = (so0, so1)

    def start_gather(ch):
        b = ch % 2
        return pltpu.async_copy(
            table_hbm.at[idx_v.at[pl.ds(ch * _C, _C)]], bufs[b], gsems[b]
        )

    def start_out(ch):
        b = ch % 2
        cps = []
        for k in range(_CS):
            cps.append(pltpu.async_copy(
                bufs[b].at[pl.ds(k * _L, _L)],
                out_hbm.at[sent0 + ch * _CS + k],
                osems[b],
            ))
        return cps

    gathers = [None] * _NCHUNK
    outs = [None] * _NCHUNK
    gathers[0] = start_gather(0)

    for ch in range(_NCHUNK):
        b = ch % 2
        gathers[ch].wait()
        if ch + 1 < _NCHUNK:
            if ch >= 1:
                for cp in outs[ch - 1]:
                    cp.wait()
            gathers[ch + 1] = start_gather(ch + 1)
        _scale_chunk(bufs[b])
        outs[ch] = start_out(ch)

    for cp in outs[_NCHUNK - 2]:
        cp.wait()
    for cp in outs[_NCHUNK - 1]:
        cp.wait()


@jax.jit
def _sc_embed(x_flat, table):
    mesh = plsc.VectorSubcoreMesh(core_axis_name="c", subcore_axis_name="s")
    f = pl.kernel(
        _body,
        out_type=jax.ShapeDtypeStruct((_SENT, _L, _D), jnp.float32),
        mesh=mesh,
        scratch_types=[
            pltpu.VMEM((_ROWS_PER_W,), jnp.int32),
            pltpu.VMEM((_C, _D), jnp.float32),
            pltpu.VMEM((_C, _D), jnp.float32),
            pltpu.SemaphoreType.DMA,
            pltpu.SemaphoreType.DMA,
            pltpu.SemaphoreType.DMA,
            pltpu.SemaphoreType.DMA,
        ],
        compiler_params=pltpu.CompilerParams(use_tc_tiling_on_sc=True),
    )
    return f(x_flat, table)


def kernel(x, table):
    x_flat = x.reshape(-1)
    return _sc_embed(x_flat, table)


# speedup vs baseline: 8.8175x; 1.7247x over previous
import math

import jax
import jax.numpy as jnp
from jax import lax
from jax.experimental import pallas as pl
from jax.experimental.pallas import tpu as pltpu
from jax.experimental.pallas import tpu_sc as plsc

_VOCAB = 100000
_D = 128
_SCALE = math.sqrt(float(_D))

_NC = 2
_NS = 16
_NW = _NC * _NS

_B = 4096 * 50
_ROWS_PER_W = _B // _NW
_C = 400
_NCHUNK = _ROWS_PER_W // _C


def _scale_chunk(rows_v):
    @plsc.parallel_loop(0, _C, step=1, unroll=4)
    def _(r):
        for k in range(_D // 16):
            sl = rows_v[r, pl.ds(k * 16, 16)]
            rows_v[r, pl.ds(k * 16, 16)] = sl * _SCALE


def _body(x_hbm, table_hbm, out_hbm, idx_v, rows0, rows1, sg0, sg1, so0, so1):
    c = lax.axis_index("c")
    s = lax.axis_index("s")
    wid = s * _NC + c
    base = wid * _ROWS_PER_W

    pltpu.sync_copy(x_hbm.at[pl.ds(base, _ROWS_PER_W)], idx_v)

    bufs = (rows0, rows1)
    gsems = (sg0, sg1)
    osems = (so0, so1)

    def start_gather(ch):
        b = ch % 2
        return pltpu.async_copy(
            table_hbm.at[idx_v.at[pl.ds(ch * _C, _C)]], bufs[b], gsems[b]
        )

    gathers = [None] * _NCHUNK
    outs = [None] * _NCHUNK
    gathers[0] = start_gather(0)

    for ch in range(_NCHUNK):
        b = ch % 2
        gathers[ch].wait()
        if ch + 1 < _NCHUNK:
            if ch >= 1:
                outs[ch - 1].wait()
            gathers[ch + 1] = start_gather(ch + 1)
        _scale_chunk(bufs[b])
        outs[ch] = pltpu.async_copy(
            bufs[b], out_hbm.at[pl.ds(base + ch * _C, _C)], osems[b]
        )

    outs[_NCHUNK - 2].wait()
    outs[_NCHUNK - 1].wait()


@jax.jit
def _sc_embed(x_flat, table):
    mesh = plsc.VectorSubcoreMesh(core_axis_name="c", subcore_axis_name="s")
    f = pl.kernel(
        _body,
        out_type=jax.ShapeDtypeStruct((_B, _D), jnp.float32),
        mesh=mesh,
        scratch_types=[
            pltpu.VMEM((_ROWS_PER_W,), jnp.int32),
            pltpu.VMEM((_C, _D), jnp.float32),
            pltpu.VMEM((_C, _D), jnp.float32),
            pltpu.SemaphoreType.DMA,
            pltpu.SemaphoreType.DMA,
            pltpu.SemaphoreType.DMA,
            pltpu.SemaphoreType.DMA,
        ],
        compiler_params=pltpu.CompilerParams(use_tc_tiling_on_sc=True),
    )
    return f(x_flat, table)


def kernel(x, table):
    n, l = x.shape
    x_flat = x.T.reshape(-1)
    out = _sc_embed(x_flat, table)
    return out.reshape(l, n, _D).transpose(1, 0, 2)
